# baseline (speedup 1.0000x reference)
import jax
import jax.numpy as jnp
from jax import lax
from jax.experimental import pallas as pl
from jax.experimental.pallas import tpu as pltpu


_NQ = 2
_HM = 200

_DN_T = (((1,), (1,)), ((), ()))


def _body(*refs):
    l_refs = refs[:_NQ]
    f_ref, w1_ref, w2_ref, b1_ref, b2_ref, out_ref, fkb_ref = refs[_NQ:]
    hm = l_refs[0].shape[0]
    m = pl.program_id(0)

    @pl.when(m == 0)
    def _():
        fkb_ref[...] = f_ref[...].astype(jnp.bfloat16)

    fkb = fkb_ref[...]
    w1 = w1_ref[...].astype(jnp.bfloat16)
    w2 = w2_ref[...].astype(jnp.bfloat16)
    b = (b1_ref[...] + b2_ref[...]).reshape(1, -1)
    for i, l_ref in enumerate(l_refs):
        x = jnp.dot(l_ref[...].astype(jnp.bfloat16), fkb,
                    preferred_element_type=jnp.float32)
        f = f_ref[pl.ds(m * _NQ * hm + i * hm, hm), :]
        out_ref[pl.ds(i * hm, hm), :] = (
            lax.dot_general((f + x).astype(jnp.bfloat16), w1, _DN_T,
                            preferred_element_type=jnp.float32)
            + lax.dot_general((x * f).astype(jnp.bfloat16), w2, _DN_T,
                              preferred_element_type=jnp.float32)
            + b
        )


def kernel(lap_matrix, eye_matrix, features, W1, b1, W2, b2):
    del eye_matrix
    n, d = features.shape
    g = n // (_NQ * _HM)

    out = pl.pallas_call(
        _body,
        grid=(g,),
        in_specs=[
            pl.BlockSpec((_HM, n), lambda m, j=j: (_NQ * m + j, 0))
            for j in range(_NQ)
        ] + [
            pl.BlockSpec((n, d), lambda m: (0, 0)),
            pl.BlockSpec((d, d), lambda m: (0, 0)),
            pl.BlockSpec((d, d), lambda m: (0, 0)),
            pl.BlockSpec((d,), lambda m: (0,)),
            pl.BlockSpec((d,), lambda m: (0,)),
        ],
        out_specs=pl.BlockSpec((_NQ * _HM, d), lambda m: (m, 0)),
        out_shape=jax.ShapeDtypeStruct((n, d), jnp.float32),
        scratch_shapes=[pltpu.VMEM((n, d), jnp.bfloat16)],
        compiler_params=pltpu.CompilerParams(
            dimension_semantics=("arbitrary",),
        ),
    )(*([lap_matrix] * _NQ), features, W1, W2, b1, b2)
    return out
